# Initial kernel scaffold; baseline (speedup 1.0000x reference)
import jax
import jax.numpy as jnp
from jax.experimental import pallas as pl


def kernel(indices, emb_table, lin_table, W1, b1, W2, b2, W3, b3):
    raise NotImplementedError("write your pallas kernel here")


# speedup vs baseline: 1.3268x; 1.3268x over previous
import functools

import jax
import jax.numpy as jnp
from jax import lax
from jax.experimental import pallas as pl
from jax.experimental.pallas import tpu as pltpu
from jax.experimental.pallas import tpu_sc as plsc

B = 16384
F = 26
V = 1000000
D = 16
H1 = 400
H2 = 400

N = B * F
NC = 2
NS = 16
NW = NC * NS
N_PER_W = N // NW
CHUNK = 1664
N_CHUNKS = N_PER_W // CHUNK


def _sc_gather_body(idx_hbm, emb_hbm, lin_hbm, emb_out, lin_out,
                    idx_v, emb_v, lin_v, sem_e, sem_l):
    wid = lax.axis_index("s") * NC + lax.axis_index("c")
    base = wid * N_PER_W
    for c in range(N_CHUNKS):
        off = base + c * CHUNK
        pltpu.sync_copy(idx_hbm.at[pl.ds(off, CHUNK)], idx_v)
        cp_e = pltpu.async_copy(emb_hbm.at[idx_v], emb_v, sem_e)
        cp_l = pltpu.async_copy(lin_hbm.at[idx_v], lin_v, sem_l)
        cp_e.wait()
        cp_l.wait()
        pltpu.sync_copy(emb_v, emb_out.at[pl.ds(off, CHUNK)])
        pltpu.sync_copy(lin_v, lin_out.at[pl.ds(off, CHUNK)])


@functools.partial(jax.jit, donate_argnums=())
def _sc_gather(idx_flat, emb_table, lin_flat):
    mesh = plsc.VectorSubcoreMesh(core_axis_name="c", subcore_axis_name="s")
    return pl.kernel(
        _sc_gather_body,
        out_type=[
            jax.ShapeDtypeStruct((N, D), jnp.float32),
            jax.ShapeDtypeStruct((N,), jnp.float32),
        ],
        mesh=mesh,
        compiler_params=pltpu.CompilerParams(use_tc_tiling_on_sc=False),
        scratch_types=[
            pltpu.VMEM((CHUNK,), jnp.int32),
            pltpu.VMEM((CHUNK, D), jnp.float32),
            pltpu.VMEM((CHUNK,), jnp.float32),
            pltpu.SemaphoreType.DMA,
            pltpu.SemaphoreType.DMA,
        ],
    )(idx_flat, emb_table, lin_flat)


BLK = 1024


def _tc_body(emb_ref, lin_ref, w1_ref, b1_ref, w2_ref, b2_ref, w3_ref,
             b3_ref, s_ref, out_ref):
    emb = emb_ref[...]
    s = s_ref[...]
    sum_emb = jnp.dot(emb, s, preferred_element_type=jnp.float32)
    sum_sq = jnp.dot(emb * emb, s, preferred_element_type=jnp.float32)
    fm = 0.5 * jnp.sum(sum_emb * sum_emb - sum_sq, axis=-1, keepdims=True)
    first = jnp.sum(lin_ref[...], axis=-1, keepdims=True)
    h = jnp.dot(emb, w1_ref[...], preferred_element_type=jnp.float32)
    h = jnp.maximum(h + b1_ref[...], 0.0)
    h = jnp.dot(h, w2_ref[...], preferred_element_type=jnp.float32)
    h = jnp.maximum(h + b2_ref[...], 0.0)
    dnn = jnp.sum(h * w3_ref[...], axis=-1, keepdims=True) + b3_ref[...]
    out_ref[...] = jax.nn.sigmoid(first + fm + dnn)


def _tc_head(emb_flat, lin_vals, W1, b1, W2, b2, W3, b3, s_mat):
    grid = (B // BLK,)
    return pl.pallas_call(
        _tc_body,
        grid=grid,
        in_specs=[
            pl.BlockSpec((BLK, F * D), lambda i: (i, 0)),
            pl.BlockSpec((BLK, F), lambda i: (i, 0)),
            pl.BlockSpec((F * D, H1), lambda i: (0, 0)),
            pl.BlockSpec((1, H1), lambda i: (0, 0)),
            pl.BlockSpec((H1, H2), lambda i: (0, 0)),
            pl.BlockSpec((1, H2), lambda i: (0, 0)),
            pl.BlockSpec((1, H2), lambda i: (0, 0)),
            pl.BlockSpec((1, 1), lambda i: (0, 0)),
            pl.BlockSpec((F * D, D), lambda i: (0, 0)),
        ],
        out_specs=pl.BlockSpec((BLK, 1), lambda i: (i, 0)),
        out_shape=jax.ShapeDtypeStruct((B, 1), jnp.float32),
    )(emb_flat, lin_vals, W1, b1, W2, b2, W3, b3, s_mat)


def kernel(indices, emb_table, lin_table, W1, b1, W2, b2, W3, b3):
    idx_flat = indices.reshape(-1).astype(jnp.int32)
    lin_flat = lin_table.reshape(-1)
    emb_rows, lin_rows = _sc_gather(idx_flat, emb_table, lin_flat)
    emb_flat = emb_rows.reshape(B, F * D)
    lin_vals = lin_rows.reshape(B, F)
    s_mat = jnp.tile(jnp.eye(D, dtype=jnp.float32), (F, 1))
    return _tc_head(emb_flat, lin_vals, W1, b1.reshape(1, H1), W2,
                    b2.reshape(1, H2), W3.reshape(1, H2), b3.reshape(1, 1),
                    s_mat)


# speedup vs baseline: 1.3289x; 1.0016x over previous
import functools

import jax
import jax.numpy as jnp
from jax import lax
from jax.experimental import pallas as pl
from jax.experimental.pallas import tpu as pltpu
from jax.experimental.pallas import tpu_sc as plsc

B = 16384
F = 26
V = 1000000
D = 16
H1 = 400
H2 = 400

N = B * F
NC = 2
NS = 16
NW = NC * NS
N_PER_W = N // NW
CHUNK = 1664
N_CHUNKS = N_PER_W // CHUNK


def _sc_gather_body(idx_hbm, emb_hbm, lin_hbm, emb_out, lin_out,
                    idx_v, emb_v, lin_v, sem_e, sem_l):
    wid = lax.axis_index("s") * NC + lax.axis_index("c")
    base = wid * N_PER_W
    for c in range(N_CHUNKS):
        off = base + c * CHUNK
        pltpu.sync_copy(idx_hbm.at[pl.ds(off, CHUNK)], idx_v)
        cp_e = pltpu.async_copy(emb_hbm.at[idx_v], emb_v, sem_e)
        cp_l = pltpu.async_copy(lin_hbm.at[idx_v], lin_v, sem_l)
        cp_e.wait()
        cp_l.wait()
        pltpu.sync_copy(emb_v, emb_out.at[pl.ds(off, CHUNK)])
        pltpu.sync_copy(lin_v, lin_out.at[pl.ds(off, CHUNK)])


@functools.partial(jax.jit, donate_argnums=())
def _sc_gather(idx_flat, emb_table, lin_flat):
    mesh = plsc.VectorSubcoreMesh(core_axis_name="c", subcore_axis_name="s")
    return pl.kernel(
        _sc_gather_body,
        out_type=[
            jax.ShapeDtypeStruct((N, D), jnp.float32),
            jax.ShapeDtypeStruct((N,), jnp.float32),
        ],
        name="deepfm_sc_gather",
        mesh=mesh,
        compiler_params=pltpu.CompilerParams(use_tc_tiling_on_sc=False),
        scratch_types=[
            pltpu.VMEM((CHUNK,), jnp.int32),
            pltpu.VMEM((CHUNK, D), jnp.float32),
            pltpu.VMEM((CHUNK,), jnp.float32),
            pltpu.SemaphoreType.DMA,
            pltpu.SemaphoreType.DMA,
        ],
    )(idx_flat, emb_table, lin_flat)


BLK = 1024


def _tc_body(emb_ref, lin_ref, w1_ref, b1_ref, w2_ref, b2_ref, w3_ref,
             b3_ref, s_ref, out_ref):
    emb = emb_ref[...]
    s = s_ref[...]
    sum_emb = jnp.dot(emb, s, preferred_element_type=jnp.float32)
    sum_sq = jnp.dot(emb * emb, s, preferred_element_type=jnp.float32)
    fm = 0.5 * jnp.sum(sum_emb * sum_emb - sum_sq, axis=-1, keepdims=True)
    first = jnp.sum(lin_ref[...], axis=-1, keepdims=True)
    h = jnp.dot(emb, w1_ref[...], preferred_element_type=jnp.float32)
    h = jnp.maximum(h + b1_ref[...], 0.0)
    h = jnp.dot(h, w2_ref[...], preferred_element_type=jnp.float32)
    h = jnp.maximum(h + b2_ref[...], 0.0)
    dnn = jnp.sum(h * w3_ref[...], axis=-1, keepdims=True) + b3_ref[...]
    out_ref[...] = jax.nn.sigmoid(first + fm + dnn)


def _tc_head(emb_flat, lin_vals, W1, b1, W2, b2, W3, b3, s_mat):
    grid = (B // BLK,)
    return pl.pallas_call(
        _tc_body,
        grid=grid,
        in_specs=[
            pl.BlockSpec((BLK, F * D), lambda i: (i, 0)),
            pl.BlockSpec((BLK, F), lambda i: (i, 0)),
            pl.BlockSpec((F * D, H1), lambda i: (0, 0)),
            pl.BlockSpec((1, H1), lambda i: (0, 0)),
            pl.BlockSpec((H1, H2), lambda i: (0, 0)),
            pl.BlockSpec((1, H2), lambda i: (0, 0)),
            pl.BlockSpec((1, H2), lambda i: (0, 0)),
            pl.BlockSpec((1, 1), lambda i: (0, 0)),
            pl.BlockSpec((F * D, D), lambda i: (0, 0)),
        ],
        out_specs=pl.BlockSpec((BLK, 1), lambda i: (i, 0)),
        out_shape=jax.ShapeDtypeStruct((B, 1), jnp.float32),
    )(emb_flat, lin_vals, W1, b1, W2, b2, W3, b3, s_mat)


def kernel(indices, emb_table, lin_table, W1, b1, W2, b2, W3, b3):
    idx_flat = indices.reshape(-1).astype(jnp.int32)
    lin_flat = lin_table.reshape(-1)
    emb_r8 = lax.optimization_barrier(emb_table.reshape(V // 8, D * 8))
    emb_rows, lin_rows = _sc_gather(idx_flat, emb_r8.reshape(V, D), lin_flat)
    emb_flat = emb_rows.reshape(B, F * D)
    lin_vals = lin_rows.reshape(B, F)
    s_mat = jnp.tile(jnp.eye(D, dtype=jnp.float32), (F, 1))
    return _tc_head(emb_flat, lin_vals, W1, b1.reshape(1, H1), W2,
                    b2.reshape(1, H2), W3.reshape(1, H2), b3.reshape(1, 1),
                    s_mat)


# speedup vs baseline: 1.3290x; 1.0001x over previous
import functools

import jax
import jax.numpy as jnp
from jax import lax
from jax.experimental import pallas as pl
from jax.experimental.pallas import tpu as pltpu
from jax.experimental.pallas import tpu_sc as plsc

B = 16384
F = 26
V = 1000000
D = 16
H1 = 400
H2 = 400

N = B * F
NC = 2
NS = 16
NW = NC * NS
N_PER_W = N // NW
CHUNK = 1664
N_CHUNKS = N_PER_W // CHUNK


NB = (V // 128)
NB_LO = NB // NW
NB_EXTRA = NB - NB_LO * NW


def _sc_transpose_body(embt_hbm, tail_hbm, z_out, in_v, out_v, tail_v):
    wid = lax.axis_index("s") * NC + lax.axis_index("c")
    iota = lax.iota(jnp.int32, (16,))
    cnt = jnp.where(wid < NB_EXTRA, NB_LO + 1, NB_LO)
    start = NB_LO * wid + jnp.minimum(wid, NB_EXTRA)

    def blk_body(i, _):
        gb = start + i
        pltpu.sync_copy(embt_hbm.at[:, pl.ds(gb * 128, 128)], in_v)

        def row_body(r, _):
            for j in range(8):
                col = jnp.full((16,), r * 8 + j, jnp.int32)
                v = plsc.load_gather(in_v, [iota, col])
                out_v[r, pl.ds(j * 16, 16)] = v
            return 0

        lax.fori_loop(0, 16, row_body, 0)
        pltpu.sync_copy(out_v, z_out.at[pl.ds(gb * 16, 16)])
        return 0

    lax.fori_loop(0, cnt, blk_body, 0)

    @pl.when(wid == NW - 1)
    def _():
        pltpu.sync_copy(tail_hbm, tail_v)
        pltpu.sync_copy(tail_v, z_out.at[pl.ds(NB * 16, 8)])


@jax.jit
def _sc_transpose(emb_t, tail8):
    mesh = plsc.VectorSubcoreMesh(core_axis_name="c", subcore_axis_name="s")
    return pl.kernel(
        _sc_transpose_body,
        out_type=jax.ShapeDtypeStruct((V // 8, 128), jnp.float32),
        name="deepfm_sc_transpose",
        mesh=mesh,
        compiler_params=pltpu.CompilerParams(use_tc_tiling_on_sc=True),
        scratch_types=[
            pltpu.VMEM((16, 128), jnp.float32),
            pltpu.VMEM((16, 128), jnp.float32),
            pltpu.VMEM((8, 128), jnp.float32),
        ],
    )(emb_t, tail8)


def _sc_gather_body(idx_hbm, emb_hbm, lin_hbm, emb_out, lin_out,
                    idx_v, emb_v, lin_v, sem_e, sem_l):
    wid = lax.axis_index("s") * NC + lax.axis_index("c")
    base = wid * N_PER_W
    for c in range(N_CHUNKS):
        off = base + c * CHUNK
        pltpu.sync_copy(idx_hbm.at[pl.ds(off, CHUNK)], idx_v)
        cp_e = pltpu.async_copy(emb_hbm.at[idx_v], emb_v, sem_e)
        cp_l = pltpu.async_copy(lin_hbm.at[idx_v], lin_v, sem_l)
        cp_e.wait()
        cp_l.wait()
        pltpu.sync_copy(emb_v, emb_out.at[pl.ds(off, CHUNK)])
        pltpu.sync_copy(lin_v, lin_out.at[pl.ds(off, CHUNK)])


@functools.partial(jax.jit, donate_argnums=())
def _sc_gather(idx_flat, emb_table, lin_flat):
    mesh = plsc.VectorSubcoreMesh(core_axis_name="c", subcore_axis_name="s")
    return pl.kernel(
        _sc_gather_body,
        out_type=[
            jax.ShapeDtypeStruct((N, D), jnp.float32),
            jax.ShapeDtypeStruct((N,), jnp.float32),
        ],
        name="deepfm_sc_gather",
        mesh=mesh,
        compiler_params=pltpu.CompilerParams(use_tc_tiling_on_sc=False),
        scratch_types=[
            pltpu.VMEM((CHUNK,), jnp.int32),
            pltpu.VMEM((CHUNK, D), jnp.float32),
            pltpu.VMEM((CHUNK,), jnp.float32),
            pltpu.SemaphoreType.DMA,
            pltpu.SemaphoreType.DMA,
        ],
    )(idx_flat, emb_table, lin_flat)


BLK = 1024


def _tc_body(emb_ref, lin_ref, w1_ref, b1_ref, w2_ref, b2_ref, w3_ref,
             b3_ref, s_ref, out_ref):
    emb = emb_ref[...]
    s = s_ref[...]
    sum_emb = jnp.dot(emb, s, preferred_element_type=jnp.float32)
    sum_sq = jnp.dot(emb * emb, s, preferred_element_type=jnp.float32)
    fm = 0.5 * jnp.sum(sum_emb * sum_emb - sum_sq, axis=-1, keepdims=True)
    first = jnp.sum(lin_ref[...], axis=-1, keepdims=True)
    h = jnp.dot(emb, w1_ref[...], preferred_element_type=jnp.float32)
    h = jnp.maximum(h + b1_ref[...], 0.0)
    h = jnp.dot(h, w2_ref[...], preferred_element_type=jnp.float32)
    h = jnp.maximum(h + b2_ref[...], 0.0)
    dnn = jnp.sum(h * w3_ref[...], axis=-1, keepdims=True) + b3_ref[...]
    out_ref[...] = jax.nn.sigmoid(first + fm + dnn)


def _tc_head(emb_flat, lin_vals, W1, b1, W2, b2, W3, b3, s_mat):
    grid = (B // BLK,)
    return pl.pallas_call(
        _tc_body,
        grid=grid,
        in_specs=[
            pl.BlockSpec((BLK, F * D), lambda i: (i, 0)),
            pl.BlockSpec((BLK, F), lambda i: (i, 0)),
            pl.BlockSpec((F * D, H1), lambda i: (0, 0)),
            pl.BlockSpec((1, H1), lambda i: (0, 0)),
            pl.BlockSpec((H1, H2), lambda i: (0, 0)),
            pl.BlockSpec((1, H2), lambda i: (0, 0)),
            pl.BlockSpec((1, H2), lambda i: (0, 0)),
            pl.BlockSpec((1, 1), lambda i: (0, 0)),
            pl.BlockSpec((F * D, D), lambda i: (0, 0)),
        ],
        out_specs=pl.BlockSpec((BLK, 1), lambda i: (i, 0)),
        out_shape=jax.ShapeDtypeStruct((B, 1), jnp.float32),
    )(emb_flat, lin_vals, W1, b1, W2, b2, W3, b3, s_mat)


def kernel(indices, emb_table, lin_table, W1, b1, W2, b2, W3, b3):
    idx_flat = indices.reshape(-1).astype(jnp.int32)
    lin_flat = lin_table.reshape(-1)
    emb_r8 = lax.optimization_barrier(emb_table.reshape(V // 8, D * 8))
    emb_rows, lin_rows = _sc_gather(idx_flat, emb_r8.reshape(V, D), lin_flat)
    emb_flat = emb_rows.reshape(B, F * D)
    lin_vals = lin_rows.reshape(B, F)
    s_mat = jnp.tile(jnp.eye(D, dtype=jnp.float32), (F, 1))
    return _tc_head(emb_flat, lin_vals, W1, b1.reshape(1, H1), W2,
                    b2.reshape(1, H2), W3.reshape(1, H2), b3.reshape(1, 1),
                    s_mat)
